# Initial kernel scaffold; baseline (speedup 1.0000x reference)
import jax
import jax.numpy as jnp
from jax.experimental import pallas as pl


def kernel(x, edge_index, edge_attr, params):
    raise NotImplementedError("write your pallas kernel here")


# speedup vs baseline: 2.3161x; 2.3161x over previous
import functools

import jax
import jax.numpy as jnp
from jax import lax
from jax.experimental import pallas as pl
from jax.experimental.pallas import tpu as pltpu
from jax.experimental.pallas import tpu_sc as plsc

N_NODES = 10000
N_EDGES = 320000
LATENT = 128
OUT_DIM = 4

NC = 2
NS = 16
NW = NC * NS
CHUNK = 128

R_E = 512
R_N = 1000

_f32 = jnp.float32


def _ln(t, lns, lnb):
    mu = jnp.mean(t, axis=-1, keepdims=True)
    var = jnp.mean((t - mu) ** 2, axis=-1, keepdims=True)
    return (t - mu) * lax.rsqrt(var + 1e-5) * lns + lnb


def _dot(a, b):
    return jnp.dot(a, b, preferred_element_type=_f32)


def _enc_body(x_ref, w1, b1, w2, b2, w3, b3, lns, lnb, out_ref):
    t = jnp.maximum(_dot(x_ref[...], w1[...]) + b1[...], 0.0)
    t = jnp.maximum(_dot(t, w2[...]) + b2[...], 0.0)
    t = _dot(t, w3[...]) + b3[...]
    out_ref[...] = _ln(t, lns[...], lnb[...])


def _dec_body(x_ref, w1, b1, w2, b2, w3, b3, out_ref):
    t = jnp.maximum(_dot(x_ref[...], w1[...]) + b1[...], 0.0)
    t = jnp.maximum(_dot(t, w2[...]) + b2[...], 0.0)
    out_ref[...] = _dot(t, w3[...]) + b3[...]


def _edge_body(e_ref, g_ref, w1e, w2, b2, w3, b3, lns, lnb, out_ref):
    t = _dot(e_ref[...], w1e[...]) + g_ref[0] + g_ref[1]
    t = jnp.maximum(t, 0.0)
    t = jnp.maximum(_dot(t, w2[...]) + b2[...], 0.0)
    t = _dot(t, w3[...]) + b3[...]
    out_ref[...] = _ln(t, lns[...], lnb[...]) + e_ref[...]


def _node_body(h_ref, agg_ref, v1h, v1a, b1, v2, b2, v3, b3, lns, lnb, out_ref):
    agg = agg_ref[0] + agg_ref[1]
    t = _dot(h_ref[...], v1h[...]) + _dot(agg, v1a[...]) + b1[...]
    t = jnp.maximum(t, 0.0)
    t = jnp.maximum(_dot(t, v2[...]) + b2[...], 0.0)
    t = _dot(t, v3[...]) + b3[...]
    out_ref[...] = _ln(t, lns[...], lnb[...]) + h_ref[...]


def _prep_body(h_ref, w_ref, b_ref, out_ref):
    out_ref[...] = _dot(h_ref[...], w_ref[0]) + b_ref[0]


def _full(shape):
    return pl.BlockSpec(shape, lambda *_: tuple(0 for _ in shape))


def _enc(x, p, rows, blk):
    (w1, b1), (w2, b2), (w3, b3) = p["layers"]
    in_dim = w1.shape[0]
    return pl.pallas_call(
        _enc_body,
        grid=(rows // blk,),
        in_specs=[
            pl.BlockSpec((blk, in_dim), lambda i: (i, 0)),
            _full(w1.shape), _full((1, LATENT)),
            _full(w2.shape), _full((1, LATENT)),
            _full(w3.shape), _full((1, LATENT)),
            _full((1, LATENT)), _full((1, LATENT)),
        ],
        out_specs=pl.BlockSpec((blk, LATENT), lambda i: (i, 0)),
        out_shape=jax.ShapeDtypeStruct((rows, LATENT), _f32),
    )(x, w1, b1.reshape(1, -1), w2, b2.reshape(1, -1), w3, b3.reshape(1, -1),
      p["ln_scale"].reshape(1, -1), p["ln_bias"].reshape(1, -1))


def _dec(h, p):
    (w1, b1), (w2, b2), (w3, b3) = p["layers"]
    return pl.pallas_call(
        _dec_body,
        grid=(N_NODES // R_N,),
        in_specs=[
            pl.BlockSpec((R_N, LATENT), lambda i: (i, 0)),
            _full(w1.shape), _full((1, LATENT)),
            _full(w2.shape), _full((1, LATENT)),
            _full(w3.shape), _full((1, OUT_DIM)),
        ],
        out_specs=pl.BlockSpec((R_N, OUT_DIM), lambda i: (i, 0)),
        out_shape=jax.ShapeDtypeStruct((N_NODES, OUT_DIM), _f32),
    )(h, w1, b1.reshape(1, -1), w2, b2.reshape(1, -1), w3, b3.reshape(1, -1))


def _prep(h, w_stack, b_stack):
    nb = N_NODES // R_N
    return pl.pallas_call(
        _prep_body,
        grid=(2, nb),
        in_specs=[
            pl.BlockSpec((R_N, LATENT), lambda i, j: (j, 0)),
            pl.BlockSpec((1, LATENT, LATENT), lambda i, j: (i, 0, 0)),
            pl.BlockSpec((1, 1, LATENT), lambda i, j: (i, 0, 0)),
        ],
        out_specs=pl.BlockSpec((R_N, LATENT), lambda i, j: (i * nb + j, 0)),
        out_shape=jax.ShapeDtypeStruct((2 * N_NODES, LATENT), _f32),
    )(h, w_stack, b_stack)


def _edge_mlp(e, g3, w1e, w2, b2, w3, b3, lns, lnb):
    return pl.pallas_call(
        _edge_body,
        grid=(N_EDGES // R_E,),
        in_specs=[
            pl.BlockSpec((R_E, LATENT), lambda i: (i, 0)),
            pl.BlockSpec((2, R_E, LATENT), lambda i: (0, i, 0)),
            _full((LATENT, LATENT)), _full((LATENT, LATENT)), _full((1, LATENT)),
            _full((LATENT, LATENT)), _full((1, LATENT)),
            _full((1, LATENT)), _full((1, LATENT)),
        ],
        out_specs=pl.BlockSpec((R_E, LATENT), lambda i: (i, 0)),
        out_shape=jax.ShapeDtypeStruct((N_EDGES, LATENT), _f32),
    )(e, g3, w1e, w2, b2.reshape(1, -1), w3, b3.reshape(1, -1),
      lns.reshape(1, -1), lnb.reshape(1, -1))


def _node_mlp(h, parts, v1h, v1a, b1, v2, b2, v3, b3, lns, lnb):
    return pl.pallas_call(
        _node_body,
        grid=(N_NODES // R_N,),
        in_specs=[
            pl.BlockSpec((R_N, LATENT), lambda i: (i, 0)),
            pl.BlockSpec((2, R_N, LATENT), lambda i: (0, i, 0)),
            _full((LATENT, LATENT)), _full((LATENT, LATENT)), _full((1, LATENT)),
            _full((LATENT, LATENT)), _full((1, LATENT)),
            _full((LATENT, LATENT)), _full((1, LATENT)),
            _full((1, LATENT)), _full((1, LATENT)),
        ],
        out_specs=pl.BlockSpec((R_N, LATENT), lambda i: (i, 0)),
        out_shape=jax.ShapeDtypeStruct((N_NODES, LATENT), _f32),
    )(h, parts, v1h, v1a, b1.reshape(1, -1), v2, b2.reshape(1, -1),
      v3, b3.reshape(1, -1), lns.reshape(1, -1), lnb.reshape(1, -1))


@functools.cache
def _get_gather_k():
    mesh = plsc.VectorSubcoreMesh(core_axis_name="c", subcore_axis_name="s",
                                  num_cores=NC, num_subcores=NS)

    @functools.partial(
        pl.kernel,
        mesh=mesh,
        out_type=jax.ShapeDtypeStruct((2 * N_EDGES, LATENT), _f32),
        scratch_types=[
            pltpu.VMEM((CHUNK,), jnp.int32),
            pltpu.VMEM((CHUNK, LATENT), _f32),
            pltpu.SemaphoreType.DMA,
        ],
    )
    def _gather_k(table_hbm, idx_hbm, out_hbm, idx_v, rows_v, sem):
        wid = lax.axis_index("s") * NC + lax.axis_index("c")
        total_chunks = (2 * N_EDGES) // CHUNK
        base_n = total_chunks // NW
        extra = total_chunks - base_n * NW
        n_mine = base_n + jnp.where(wid < extra, 1, 0)

        def body(t, carry):
            base = (wid + NW * t) * CHUNK
            pltpu.sync_copy(idx_hbm.at[pl.ds(base, CHUNK)], idx_v)
            pltpu.async_copy(table_hbm.at[idx_v], rows_v, sem).wait()
            pltpu.sync_copy(rows_v, out_hbm.at[pl.ds(base, CHUNK)])
            return carry

        lax.fori_loop(0, n_mine, body, 0)

    return _gather_k


@functools.cache
def _get_scatter_k():
    mesh = plsc.VectorSubcoreMesh(core_axis_name="c", subcore_axis_name="s",
                                  num_cores=NC, num_subcores=NS)

    @functools.partial(
        pl.kernel,
        mesh=mesh,
        out_type=jax.ShapeDtypeStruct((NC, N_NODES, LATENT), _f32),
        scratch_types=[
            pltpu.VMEM((CHUNK,), jnp.int32),
            pltpu.VMEM((CHUNK, LATENT), _f32),
            pltpu.VMEM_SHARED((N_NODES, LATENT), _f32),
            pltpu.SemaphoreType.DMA,
        ],
    )
    def _scatter_k(e_hbm, dst_hbm, z_hbm, out_hbm, idx_v, rows_v, acc_sh, sem):
        c = lax.axis_index("c")
        s = lax.axis_index("s")
        wid = s * NC + c
        stripe = 624
        tail = N_NODES - NS * stripe
        pltpu.sync_copy(z_hbm.at[pl.ds(s * stripe, stripe)],
                        acc_sh.at[pl.ds(s * stripe, stripe)])

        @pl.when(s == NS - 1)
        def _():
            pltpu.sync_copy(z_hbm.at[pl.ds(NS * stripe, tail)],
                            acc_sh.at[pl.ds(NS * stripe, tail)])

        plsc.subcore_barrier()

        total_chunks = N_EDGES // CHUNK
        base_n = total_chunks // NW
        extra = total_chunks - base_n * NW
        n_mine = base_n + jnp.where(wid < extra, 1, 0)

        def body(t, carry):
            base = (wid + NW * t) * CHUNK
            pltpu.sync_copy(dst_hbm.at[pl.ds(base, CHUNK)], idx_v)
            pltpu.sync_copy(e_hbm.at[pl.ds(base, CHUNK)], rows_v)
            pltpu.sync_copy(rows_v, acc_sh.at[idx_v], add=True)
            return carry

        lax.fori_loop(0, n_mine, body, 0)
        plsc.subcore_barrier()
        pltpu.sync_copy(acc_sh.at[pl.ds(s * stripe, stripe)],
                        out_hbm.at[c, pl.ds(s * stripe, stripe)])

        @pl.when(s == NS - 1)
        def _():
            pltpu.sync_copy(acc_sh.at[pl.ds(NS * stripe, tail)],
                            out_hbm.at[c, pl.ds(NS * stripe, tail)])

    return _scatter_k


def kernel(x, edge_index, edge_attr, params):
    src = edge_index[0]
    dst = edge_index[1]
    idx_all = jnp.concatenate([src, dst + N_NODES])
    zeros_nodes = jnp.zeros((N_NODES, LATENT), _f32)

    h = _enc(x, params["node_enc"], N_NODES, R_N)
    e = _enc(edge_attr, params["edge_enc"], N_EDGES, R_E)

    for layer in params["proc"]:
        ep = layer["edge"]
        (w1, b1), (w2, b2), (w3, b3) = ep["layers"]
        w1e = w1[:LATENT]
        w_stack = jnp.stack([w1[LATENT:2 * LATENT], w1[2 * LATENT:]])
        b_stack = jnp.stack([b1.reshape(1, -1), jnp.zeros((1, LATENT), _f32)])

        table = _prep(h, w_stack, b_stack)
        g3 = _get_gather_k()(table, idx_all).reshape(2, N_EDGES, LATENT)
        e = _edge_mlp(e, g3, w1e, w2, b2, w3, b3, ep["ln_scale"], ep["ln_bias"])

        parts = _get_scatter_k()(e, dst, zeros_nodes)

        np_ = layer["node"]
        (v1, nb1), (v2, nb2), (v3, nb3) = np_["layers"]
        h = _node_mlp(h, parts, v1[:LATENT], v1[LATENT:], nb1, v2, nb2,
                      v3, nb3, np_["ln_scale"], np_["ln_bias"])

    return _dec(h, params["dec"])


# speedup vs baseline: 2.4462x; 1.0562x over previous
import functools

import jax
import jax.numpy as jnp
from jax import lax
from jax.experimental import pallas as pl
from jax.experimental.pallas import tpu as pltpu
from jax.experimental.pallas import tpu_sc as plsc

N_NODES = 10000
N_EDGES = 320000
LATENT = 128
OUT_DIM = 4

NC = 2
NS = 16
NW = NC * NS
CHUNK = 128
HALF = N_NODES // 2

R_E = 512
R_N = 1000

_f32 = jnp.float32


def _ln(t, lns, lnb):
    mu = jnp.mean(t, axis=-1, keepdims=True)
    var = jnp.mean((t - mu) ** 2, axis=-1, keepdims=True)
    return (t - mu) * lax.rsqrt(var + 1e-5) * lns + lnb


def _dot(a, b):
    return jnp.dot(a, b, preferred_element_type=_f32)


def _enc_body(x_ref, w1, b1, w2, b2, w3, b3, lns, lnb, out_ref):
    t = jnp.maximum(_dot(x_ref[...], w1[...]) + b1[...], 0.0)
    t = jnp.maximum(_dot(t, w2[...]) + b2[...], 0.0)
    t = _dot(t, w3[...]) + b3[...]
    out_ref[...] = _ln(t, lns[...], lnb[...])


def _dec_body(x_ref, w1, b1, w2, b2, w3, b3, out_ref):
    t = jnp.maximum(_dot(x_ref[...], w1[...]) + b1[...], 0.0)
    t = jnp.maximum(_dot(t, w2[...]) + b2[...], 0.0)
    out_ref[...] = _dot(t, w3[...]) + b3[...]


def _edge_body(e_ref, g_ref, w1e, w2, b2, w3, b3, lns, lnb, out_ref):
    t = _dot(e_ref[...], w1e[...]) + g_ref[0] + g_ref[1]
    t = jnp.maximum(t, 0.0)
    t = jnp.maximum(_dot(t, w2[...]) + b2[...], 0.0)
    t = _dot(t, w3[...]) + b3[...]
    out_ref[...] = _ln(t, lns[...], lnb[...]) + e_ref[...]


def _node_body(h_ref, agg_ref, v1h, v1a, b1, v2, b2, v3, b3, lns, lnb, out_ref):
    t = _dot(h_ref[...], v1h[...]) + _dot(agg_ref[...], v1a[...]) + b1[...]
    t = jnp.maximum(t, 0.0)
    t = jnp.maximum(_dot(t, v2[...]) + b2[...], 0.0)
    t = _dot(t, v3[...]) + b3[...]
    out_ref[...] = _ln(t, lns[...], lnb[...]) + h_ref[...]


def _prep_body(h_ref, w_ref, b_ref, out_ref):
    out_ref[...] = _dot(h_ref[...], w_ref[0]) + b_ref[0]


def _full(shape):
    return pl.BlockSpec(shape, lambda *_: tuple(0 for _ in shape))


def _enc(x, p, rows, blk):
    (w1, b1), (w2, b2), (w3, b3) = p["layers"]
    in_dim = w1.shape[0]
    return pl.pallas_call(
        _enc_body,
        grid=(rows // blk,),
        in_specs=[
            pl.BlockSpec((blk, in_dim), lambda i: (i, 0)),
            _full(w1.shape), _full((1, LATENT)),
            _full(w2.shape), _full((1, LATENT)),
            _full(w3.shape), _full((1, LATENT)),
            _full((1, LATENT)), _full((1, LATENT)),
        ],
        out_specs=pl.BlockSpec((blk, LATENT), lambda i: (i, 0)),
        out_shape=jax.ShapeDtypeStruct((rows, LATENT), _f32),
    )(x, w1, b1.reshape(1, -1), w2, b2.reshape(1, -1), w3, b3.reshape(1, -1),
      p["ln_scale"].reshape(1, -1), p["ln_bias"].reshape(1, -1))


def _dec(h, p):
    (w1, b1), (w2, b2), (w3, b3) = p["layers"]
    return pl.pallas_call(
        _dec_body,
        grid=(N_NODES // R_N,),
        in_specs=[
            pl.BlockSpec((R_N, LATENT), lambda i: (i, 0)),
            _full(w1.shape), _full((1, LATENT)),
            _full(w2.shape), _full((1, LATENT)),
            _full(w3.shape), _full((1, OUT_DIM)),
        ],
        out_specs=pl.BlockSpec((R_N, OUT_DIM), lambda i: (i, 0)),
        out_shape=jax.ShapeDtypeStruct((N_NODES, OUT_DIM), _f32),
    )(h, w1, b1.reshape(1, -1), w2, b2.reshape(1, -1), w3, b3.reshape(1, -1))


def _prep(h, w_stack, b_stack):
    nb = N_NODES // R_N
    return pl.pallas_call(
        _prep_body,
        grid=(2, nb),
        in_specs=[
            pl.BlockSpec((R_N, LATENT), lambda i, j: (j, 0)),
            pl.BlockSpec((1, LATENT, LATENT), lambda i, j: (i, 0, 0)),
            pl.BlockSpec((1, 1, LATENT), lambda i, j: (i, 0, 0)),
        ],
        out_specs=pl.BlockSpec((R_N, LATENT), lambda i, j: (i * nb + j, 0)),
        out_shape=jax.ShapeDtypeStruct((2 * N_NODES, LATENT), _f32),
    )(h, w_stack, b_stack)


def _edge_mlp(e, g3, w1e, w2, b2, w3, b3, lns, lnb):
    return pl.pallas_call(
        _edge_body,
        grid=(N_EDGES // R_E,),
        in_specs=[
            pl.BlockSpec((R_E, LATENT), lambda i: (i, 0)),
            pl.BlockSpec((2, R_E, LATENT), lambda i: (0, i, 0)),
            _full((LATENT, LATENT)), _full((LATENT, LATENT)), _full((1, LATENT)),
            _full((LATENT, LATENT)), _full((1, LATENT)),
            _full((1, LATENT)), _full((1, LATENT)),
        ],
        out_specs=pl.BlockSpec((R_E, LATENT), lambda i: (i, 0)),
        out_shape=jax.ShapeDtypeStruct((N_EDGES, LATENT), _f32),
    )(e, g3, w1e, w2, b2.reshape(1, -1), w3, b3.reshape(1, -1),
      lns.reshape(1, -1), lnb.reshape(1, -1))


def _node_mlp(h, agg, v1h, v1a, b1, v2, b2, v3, b3, lns, lnb):
    return pl.pallas_call(
        _node_body,
        grid=(N_NODES // R_N,),
        in_specs=[
            pl.BlockSpec((R_N, LATENT), lambda i: (i, 0)),
            pl.BlockSpec((R_N, LATENT), lambda i: (i, 0)),
            _full((LATENT, LATENT)), _full((LATENT, LATENT)), _full((1, LATENT)),
            _full((LATENT, LATENT)), _full((1, LATENT)),
            _full((LATENT, LATENT)), _full((1, LATENT)),
            _full((1, LATENT)), _full((1, LATENT)),
        ],
        out_specs=pl.BlockSpec((R_N, LATENT), lambda i: (i, 0)),
        out_shape=jax.ShapeDtypeStruct((N_NODES, LATENT), _f32),
    )(h, agg, v1h, v1a, b1.reshape(1, -1), v2, b2.reshape(1, -1),
      v3, b3.reshape(1, -1), lns.reshape(1, -1), lnb.reshape(1, -1))


NB = 4


@functools.cache
def _get_gather_k():
    mesh = plsc.VectorSubcoreMesh(core_axis_name="c", subcore_axis_name="s",
                                  num_cores=NC, num_subcores=NS)
    scratch = ([pltpu.VMEM((CHUNK,), jnp.int32) for _ in range(NB)]
               + [pltpu.VMEM((CHUNK, LATENT), _f32) for _ in range(NB)]
               + [pltpu.SemaphoreType.DMA for _ in range(3 * NB)])

    @functools.partial(
        pl.kernel,
        mesh=mesh,
        out_type=jax.ShapeDtypeStruct((2 * N_EDGES, LATENT), _f32),
        scratch_types=scratch,
    )
    def _gather_k(table_hbm, idx_hbm, out_hbm, *bufs):
        idxb = bufs[:NB]
        rows = bufs[NB:2 * NB]
        si = bufs[2 * NB:3 * NB]
        sg = bufs[3 * NB:4 * NB]
        sw = bufs[4 * NB:5 * NB]
        wid = lax.axis_index("s") * NC + lax.axis_index("c")
        n_groups = (2 * N_EDGES) // CHUNK // NB
        base_n = n_groups // NW
        extra = n_groups - base_n * NW
        n_mine = base_n + jnp.where(wid < extra, 1, 0)

        def body(u, carry):
            g0 = (wid + NW * u) * NB
            di = [pltpu.async_copy(
                idx_hbm.at[pl.ds((g0 + b) * CHUNK, CHUNK)], idxb[b], si[b])
                for b in range(NB)]
            dg = []
            for b in range(NB):
                di[b].wait()
                dg.append(pltpu.async_copy(table_hbm.at[idxb[b]], rows[b], sg[b]))
            dw = []
            for b in range(NB):
                dg[b].wait()
                dw.append(pltpu.async_copy(
                    rows[b], out_hbm.at[pl.ds((g0 + b) * CHUNK, CHUNK)], sw[b]))
            for b in range(NB):
                dw[b].wait()
            return carry

        lax.fori_loop(0, n_mine, body, 0)

    return _gather_k


@functools.cache
def _get_scatter_k():
    mesh = plsc.VectorSubcoreMesh(core_axis_name="c", subcore_axis_name="s",
                                  num_cores=NC, num_subcores=NS)

    scratch = ([pltpu.VMEM((CHUNK,), jnp.int32) for _ in range(NB)]
               + [pltpu.VMEM((CHUNK, LATENT), _f32) for _ in range(NB)]
               + [pltpu.VMEM_SHARED((HALF + 8, LATENT), _f32)]
               + [pltpu.SemaphoreType.DMA for _ in range(3 * NB)])

    @functools.partial(
        pl.kernel,
        mesh=mesh,
        out_type=jax.ShapeDtypeStruct((NC, HALF, LATENT), _f32),
        scratch_types=scratch,
    )
    def _scatter_k(e_hbm, dst_hbm, z_hbm, out_hbm, *bufs):
        idxb = bufs[:NB]
        rows = bufs[NB:2 * NB]
        acc_sh = bufs[2 * NB]
        si = bufs[2 * NB + 1:2 * NB + 1 + NB]
        sr = bufs[2 * NB + 1 + NB:2 * NB + 1 + 2 * NB]
        sa = bufs[2 * NB + 1 + 2 * NB:2 * NB + 1 + 3 * NB]
        c = lax.axis_index("c")
        s = lax.axis_index("s")
        lo = c * HALF
        stripe = 312
        pltpu.sync_copy(z_hbm.at[pl.ds(s * stripe, stripe)],
                        acc_sh.at[pl.ds(s * stripe, stripe)])

        @pl.when(s == NS - 1)
        def _():
            pltpu.sync_copy(z_hbm.at[pl.ds(NS * stripe, 16)],
                            acc_sh.at[pl.ds(NS * stripe, 16)])

        plsc.subcore_barrier()

        n_groups = N_EDGES // CHUNK // NB
        base_n = n_groups // NS
        extra = n_groups - base_n * NS
        n_mine = base_n + jnp.where(s < extra, 1, 0)

        def body(u, carry):
            g0 = (s + NS * u) * NB
            di = [pltpu.async_copy(
                dst_hbm.at[pl.ds((g0 + b) * CHUNK, CHUNK)], idxb[b], si[b])
                for b in range(NB)]
            dr = [pltpu.async_copy(
                e_hbm.at[pl.ds((g0 + b) * CHUNK, CHUNK)], rows[b], sr[b])
                for b in range(NB)]
            da = []
            for b in range(NB):
                di[b].wait()
                for k in range(CHUNK // 16):
                    v = idxb[b][pl.ds(16 * k, 16)]
                    inb = (v >= lo) & (v < lo + HALF)
                    idxb[b][pl.ds(16 * k, 16)] = jnp.where(inb, v - lo, HALF)
                dr[b].wait()
                da.append(pltpu.async_copy(rows[b], acc_sh.at[idxb[b]], sa[b],
                                           add=True))
            for b in range(NB):
                da[b].wait()
            return carry

        lax.fori_loop(0, n_mine, body, 0)
        plsc.subcore_barrier()
        wtail = HALF - NS * stripe
        pltpu.sync_copy(acc_sh.at[pl.ds(s * stripe, stripe)],
                        out_hbm.at[c, pl.ds(s * stripe, stripe)])

        @pl.when(s == NS - 1)
        def _():
            pltpu.sync_copy(acc_sh.at[pl.ds(NS * stripe, wtail)],
                            out_hbm.at[c, pl.ds(NS * stripe, wtail)])

    return _scatter_k


def kernel(x, edge_index, edge_attr, params):
    src = edge_index[0]
    dst = edge_index[1]
    idx_all = jnp.concatenate([src, dst + N_NODES])
    zeros_nodes = jnp.zeros((N_NODES, LATENT), _f32)

    h = _enc(x, params["node_enc"], N_NODES, R_N)
    e = _enc(edge_attr, params["edge_enc"], N_EDGES, R_E)

    for layer in params["proc"]:
        ep = layer["edge"]
        (w1, b1), (w2, b2), (w3, b3) = ep["layers"]
        w1e = w1[:LATENT]
        w_stack = jnp.stack([w1[LATENT:2 * LATENT], w1[2 * LATENT:]])
        b_stack = jnp.stack([b1.reshape(1, -1), jnp.zeros((1, LATENT), _f32)])

        table = _prep(h, w_stack, b_stack)
        g3 = _get_gather_k()(table, idx_all).reshape(2, N_EDGES, LATENT)
        e = _edge_mlp(e, g3, w1e, w2, b2, w3, b3, ep["ln_scale"], ep["ln_bias"])

        agg = _get_scatter_k()(e, dst, zeros_nodes).reshape(N_NODES, LATENT)

        np_ = layer["node"]
        (v1, nb1), (v2, nb2), (v3, nb3) = np_["layers"]
        h = _node_mlp(h, agg, v1[:LATENT], v1[LATENT:], nb1, v2, nb2,
                      v3, nb3, np_["ln_scale"], np_["ln_bias"])

    return _dec(h, params["dec"])


# speedup vs baseline: 3.2957x; 1.3473x over previous
import functools

import jax
import jax.numpy as jnp
from jax import lax
from jax.experimental import pallas as pl
from jax.experimental.pallas import tpu as pltpu
from jax.experimental.pallas import tpu_sc as plsc

N_NODES = 10000
N_EDGES = 320000
LATENT = 128
OUT_DIM = 4

NC = 2
NS = 16
NW = NC * NS
CHUNK = 128
HALF = N_NODES // 2

R_E = 1600
R_N = 1000

_f32 = jnp.float32


def _ln(t, lns, lnb):
    mu = jnp.mean(t, axis=-1, keepdims=True)
    var = jnp.mean((t - mu) ** 2, axis=-1, keepdims=True)
    return (t - mu) * lax.rsqrt(var + 1e-5) * lns + lnb


def _dot(a, b):
    return jnp.dot(a, b, preferred_element_type=_f32)


def _enc_body(x_ref, w1, b1, w2, b2, w3, b3, lns, lnb, out_ref):
    t = jnp.maximum(_dot(x_ref[...], w1[...]) + b1[...], 0.0)
    t = jnp.maximum(_dot(t, w2[...]) + b2[...], 0.0)
    t = _dot(t, w3[...]) + b3[...]
    out_ref[...] = _ln(t, lns[...], lnb[...])


def _dec_body(x_ref, w1, b1, w2, b2, w3, b3, out_ref):
    t = jnp.maximum(_dot(x_ref[...], w1[...]) + b1[...], 0.0)
    t = jnp.maximum(_dot(t, w2[...]) + b2[...], 0.0)
    out_ref[...] = _dot(t, w3[...]) + b3[...]


def _edge_body(e_ref, g_ref, w1e, w2, b2, w3, b3, lns, lnb, out_ref):
    t = _dot(e_ref[...], w1e[...]) + g_ref[0] + g_ref[1]
    t = jnp.maximum(t, 0.0)
    t = jnp.maximum(_dot(t, w2[...]) + b2[...], 0.0)
    t = _dot(t, w3[...]) + b3[...]
    out_ref[...] = _ln(t, lns[...], lnb[...]) + e_ref[...]


def _node_body(h_ref, agg_ref, v1h, v1a, b1, v2, b2, v3, b3, lns, lnb, out_ref):
    t = _dot(h_ref[...], v1h[...]) + _dot(agg_ref[...], v1a[...]) + b1[...]
    t = jnp.maximum(t, 0.0)
    t = jnp.maximum(_dot(t, v2[...]) + b2[...], 0.0)
    t = _dot(t, v3[...]) + b3[...]
    out_ref[...] = _ln(t, lns[...], lnb[...]) + h_ref[...]


def _prep_body(h_ref, w_ref, b_ref, out_ref):
    out_ref[...] = _dot(h_ref[...], w_ref[0]) + b_ref[0]


def _full(shape):
    return pl.BlockSpec(shape, lambda *_: tuple(0 for _ in shape))


def _enc(x, p, rows, blk):
    (w1, b1), (w2, b2), (w3, b3) = p["layers"]
    in_dim = w1.shape[0]
    return pl.pallas_call(
        _enc_body,
        grid=(rows // blk,),
        in_specs=[
            pl.BlockSpec((blk, in_dim), lambda i: (i, 0)),
            _full(w1.shape), _full((1, LATENT)),
            _full(w2.shape), _full((1, LATENT)),
            _full(w3.shape), _full((1, LATENT)),
            _full((1, LATENT)), _full((1, LATENT)),
        ],
        out_specs=pl.BlockSpec((blk, LATENT), lambda i: (i, 0)),
        out_shape=jax.ShapeDtypeStruct((rows, LATENT), _f32),
    )(x, w1, b1.reshape(1, -1), w2, b2.reshape(1, -1), w3, b3.reshape(1, -1),
      p["ln_scale"].reshape(1, -1), p["ln_bias"].reshape(1, -1))


def _dec(h, p):
    (w1, b1), (w2, b2), (w3, b3) = p["layers"]
    return pl.pallas_call(
        _dec_body,
        grid=(N_NODES // R_N,),
        in_specs=[
            pl.BlockSpec((R_N, LATENT), lambda i: (i, 0)),
            _full(w1.shape), _full((1, LATENT)),
            _full(w2.shape), _full((1, LATENT)),
            _full(w3.shape), _full((1, OUT_DIM)),
        ],
        out_specs=pl.BlockSpec((R_N, OUT_DIM), lambda i: (i, 0)),
        out_shape=jax.ShapeDtypeStruct((N_NODES, OUT_DIM), _f32),
    )(h, w1, b1.reshape(1, -1), w2, b2.reshape(1, -1), w3, b3.reshape(1, -1))


def _prep(h, w_stack, b_stack):
    nb = N_NODES // R_N
    return pl.pallas_call(
        _prep_body,
        grid=(2, nb),
        in_specs=[
            pl.BlockSpec((R_N, LATENT), lambda i, j: (j, 0)),
            pl.BlockSpec((1, LATENT, LATENT), lambda i, j: (i, 0, 0)),
            pl.BlockSpec((1, 1, LATENT), lambda i, j: (i, 0, 0)),
        ],
        out_specs=pl.BlockSpec((R_N, LATENT), lambda i, j: (i * nb + j, 0)),
        out_shape=jax.ShapeDtypeStruct((2 * N_NODES, LATENT), _f32),
    )(h, w_stack, b_stack)


def _edge_mlp(e, g3, w1e, w2, b2, w3, b3, lns, lnb):
    return pl.pallas_call(
        _edge_body,
        grid=(N_EDGES // R_E,),
        in_specs=[
            pl.BlockSpec((R_E, LATENT), lambda i: (i, 0)),
            pl.BlockSpec((2, R_E, LATENT), lambda i: (0, i, 0)),
            _full((LATENT, LATENT)), _full((LATENT, LATENT)), _full((1, LATENT)),
            _full((LATENT, LATENT)), _full((1, LATENT)),
            _full((1, LATENT)), _full((1, LATENT)),
        ],
        out_specs=pl.BlockSpec((R_E, LATENT), lambda i: (i, 0)),
        out_shape=jax.ShapeDtypeStruct((N_EDGES, LATENT), _f32),
    )(e, g3, w1e, w2, b2.reshape(1, -1), w3, b3.reshape(1, -1),
      lns.reshape(1, -1), lnb.reshape(1, -1))


def _node_mlp(h, agg, v1h, v1a, b1, v2, b2, v3, b3, lns, lnb):
    return pl.pallas_call(
        _node_body,
        grid=(N_NODES // R_N,),
        in_specs=[
            pl.BlockSpec((R_N, LATENT), lambda i: (i, 0)),
            pl.BlockSpec((R_N, LATENT), lambda i: (i, 0)),
            _full((LATENT, LATENT)), _full((LATENT, LATENT)), _full((1, LATENT)),
            _full((LATENT, LATENT)), _full((1, LATENT)),
            _full((LATENT, LATENT)), _full((1, LATENT)),
            _full((1, LATENT)), _full((1, LATENT)),
        ],
        out_specs=pl.BlockSpec((R_N, LATENT), lambda i: (i, 0)),
        out_shape=jax.ShapeDtypeStruct((N_NODES, LATENT), _f32),
    )(h, agg, v1h, v1a, b1.reshape(1, -1), v2, b2.reshape(1, -1),
      v3, b3.reshape(1, -1), lns.reshape(1, -1), lnb.reshape(1, -1))


NB = 4


@functools.cache
def _get_gather_k():
    mesh = plsc.VectorSubcoreMesh(core_axis_name="c", subcore_axis_name="s",
                                  num_cores=NC, num_subcores=NS)
    scratch = ([pltpu.VMEM((CHUNK,), jnp.int32) for _ in range(NB)]
               + [pltpu.VMEM((CHUNK, LATENT), _f32) for _ in range(NB)]
               + [pltpu.SemaphoreType.DMA for _ in range(3 * NB)])

    @functools.partial(
        pl.kernel,
        mesh=mesh,
        out_type=jax.ShapeDtypeStruct((2 * N_EDGES, LATENT), _f32),
        scratch_types=scratch,
    )
    def _gather_k(table_hbm, idx_hbm, out_hbm, *bufs):
        idxb = bufs[:NB]
        rows = bufs[NB:2 * NB]
        si = bufs[2 * NB:3 * NB]
        sg = bufs[3 * NB:4 * NB]
        sw = bufs[4 * NB:5 * NB]
        wid = lax.axis_index("s") * NC + lax.axis_index("c")
        n_groups = (2 * N_EDGES) // CHUNK // NB
        base_n = n_groups // NW
        extra = n_groups - base_n * NW
        n_mine = base_n + jnp.where(wid < extra, 1, 0)

        def body(u, carry):
            g0 = (wid + NW * u) * NB
            di = [pltpu.async_copy(
                idx_hbm.at[pl.ds((g0 + b) * CHUNK, CHUNK)], idxb[b], si[b])
                for b in range(NB)]
            dg = []
            for b in range(NB):
                di[b].wait()
                dg.append(pltpu.async_copy(table_hbm.at[idxb[b]], rows[b], sg[b]))
            dw = []
            for b in range(NB):
                dg[b].wait()
                dw.append(pltpu.async_copy(
                    rows[b], out_hbm.at[pl.ds((g0 + b) * CHUNK, CHUNK)], sw[b]))
            for b in range(NB):
                dw[b].wait()
            return carry

        lax.fori_loop(0, n_mine, body, 0)

    return _gather_k


@functools.cache
def _get_scatter_k():
    mesh = plsc.VectorSubcoreMesh(core_axis_name="c", subcore_axis_name="s",
                                  num_cores=NC, num_subcores=NS)

    scratch = ([pltpu.VMEM((CHUNK,), jnp.int32) for _ in range(NB)]
               + [pltpu.VMEM((CHUNK, LATENT), _f32) for _ in range(NB)]
               + [pltpu.VMEM_SHARED((HALF + 8, LATENT), _f32)]
               + [pltpu.SemaphoreType.DMA for _ in range(3 * NB)])

    @functools.partial(
        pl.kernel,
        mesh=mesh,
        out_type=jax.ShapeDtypeStruct((NC, HALF, LATENT), _f32),
        scratch_types=scratch,
    )
    def _scatter_k(e_hbm, dst_hbm, z_hbm, out_hbm, *bufs):
        idxb = bufs[:NB]
        rows = bufs[NB:2 * NB]
        acc_sh = bufs[2 * NB]
        si = bufs[2 * NB + 1:2 * NB + 1 + NB]
        sr = bufs[2 * NB + 1 + NB:2 * NB + 1 + 2 * NB]
        sa = bufs[2 * NB + 1 + 2 * NB:2 * NB + 1 + 3 * NB]
        c = lax.axis_index("c")
        s = lax.axis_index("s")
        lo = c * HALF
        stripe = 312
        pltpu.sync_copy(z_hbm.at[pl.ds(s * stripe, stripe)],
                        acc_sh.at[pl.ds(s * stripe, stripe)])

        @pl.when(s == NS - 1)
        def _():
            pltpu.sync_copy(z_hbm.at[pl.ds(NS * stripe, 16)],
                            acc_sh.at[pl.ds(NS * stripe, 16)])

        plsc.subcore_barrier()

        n_groups = N_EDGES // CHUNK // NB
        base_n = n_groups // NS
        extra = n_groups - base_n * NS
        n_mine = base_n + jnp.where(s < extra, 1, 0)

        def body(u, carry):
            g0 = (s + NS * u) * NB
            di = [pltpu.async_copy(
                dst_hbm.at[pl.ds((g0 + b) * CHUNK, CHUNK)], idxb[b], si[b])
                for b in range(NB)]
            dr = [pltpu.async_copy(
                e_hbm.at[pl.ds((g0 + b) * CHUNK, CHUNK)], rows[b], sr[b])
                for b in range(NB)]
            da = []
            for b in range(NB):
                di[b].wait()
                for k in range(CHUNK // 16):
                    v = idxb[b][pl.ds(16 * k, 16)]
                    inb = (v >= lo) & (v < lo + HALF)
                    idxb[b][pl.ds(16 * k, 16)] = jnp.where(inb, v - lo, HALF)
                dr[b].wait()
                da.append(pltpu.async_copy(rows[b], acc_sh.at[idxb[b]], sa[b],
                                           add=True))
            for b in range(NB):
                da[b].wait()
            return carry

        lax.fori_loop(0, n_mine, body, 0)
        plsc.subcore_barrier()
        wtail = HALF - NS * stripe
        pltpu.sync_copy(acc_sh.at[pl.ds(s * stripe, stripe)],
                        out_hbm.at[c, pl.ds(s * stripe, stripe)])

        @pl.when(s == NS - 1)
        def _():
            pltpu.sync_copy(acc_sh.at[pl.ds(NS * stripe, wtail)],
                            out_hbm.at[c, pl.ds(NS * stripe, wtail)])

    return _scatter_k


def kernel(x, edge_index, edge_attr, params):
    src = edge_index[0]
    dst = edge_index[1]
    idx_all = jnp.concatenate([src, dst + N_NODES])
    zeros_nodes = jnp.zeros((N_NODES, LATENT), _f32)

    h = _enc(x, params["node_enc"], N_NODES, R_N)
    e = _enc(edge_attr, params["edge_enc"], N_EDGES, R_E)

    for layer in params["proc"]:
        ep = layer["edge"]
        (w1, b1), (w2, b2), (w3, b3) = ep["layers"]
        w1e = w1[:LATENT]
        w_stack = jnp.stack([w1[LATENT:2 * LATENT], w1[2 * LATENT:]])
        b_stack = jnp.stack([b1.reshape(1, -1), jnp.zeros((1, LATENT), _f32)])

        table = _prep(h, w_stack, b_stack)
        g3 = _get_gather_k()(table, idx_all).reshape(2, N_EDGES, LATENT)
        e = _edge_mlp(e, g3, w1e, w2, b2, w3, b3, ep["ln_scale"], ep["ln_bias"])

        agg = _get_scatter_k()(e, dst, zeros_nodes).reshape(N_NODES, LATENT)

        np_ = layer["node"]
        (v1, nb1), (v2, nb2), (v3, nb3) = np_["layers"]
        h = _node_mlp(h, agg, v1[:LATENT], v1[LATENT:], nb1, v2, nb2,
                      v3, nb3, np_["ln_scale"], np_["ln_bias"])

    return _dec(h, params["dec"])


# speedup vs baseline: 3.6179x; 1.0977x over previous
import functools

import jax
import jax.numpy as jnp
from jax import lax
from jax.experimental import pallas as pl
from jax.experimental.pallas import tpu as pltpu
from jax.experimental.pallas import tpu_sc as plsc

N_NODES = 10000
N_EDGES = 320000
LATENT = 128
OUT_DIM = 4

NC = 2
NS = 16
NW = NC * NS
CHUNK = 128
HALF = N_NODES // 2

R_E = 3200
R_N = 1000

_f32 = jnp.float32


def _ln(t, lns, lnb):
    mu = jnp.mean(t, axis=-1, keepdims=True)
    var = jnp.mean((t - mu) ** 2, axis=-1, keepdims=True)
    return (t - mu) * lax.rsqrt(var + 1e-5) * lns + lnb


def _dot(a, b):
    return jnp.dot(a, b, preferred_element_type=_f32)


def _enc_body(x_ref, w1, b1, w2, b2, w3, b3, lns, lnb, out_ref):
    t = jnp.maximum(_dot(x_ref[...], w1[...]) + b1[...], 0.0)
    t = jnp.maximum(_dot(t, w2[...]) + b2[...], 0.0)
    t = _dot(t, w3[...]) + b3[...]
    out_ref[...] = _ln(t, lns[...], lnb[...])


def _dec_body(x_ref, w1, b1, w2, b2, w3, b3, out_ref):
    t = jnp.maximum(_dot(x_ref[...], w1[...]) + b1[...], 0.0)
    t = jnp.maximum(_dot(t, w2[...]) + b2[...], 0.0)
    out_ref[...] = _dot(t, w3[...]) + b3[...]


def _edge_body(e_ref, g_ref, w1e, w2, b2, w3, b3, lns, lnb, out_ref):
    t = _dot(e_ref[...], w1e[...]) + g_ref[0] + g_ref[1]
    t = jnp.maximum(t, 0.0)
    t = jnp.maximum(_dot(t, w2[...]) + b2[...], 0.0)
    t = _dot(t, w3[...]) + b3[...]
    out_ref[...] = _ln(t, lns[...], lnb[...]) + e_ref[...]


def _node_body(h_ref, agg_ref, v1h, v1a, b1, v2, b2, v3, b3, lns, lnb, out_ref):
    t = _dot(h_ref[...], v1h[...]) + _dot(agg_ref[...], v1a[...]) + b1[...]
    t = jnp.maximum(t, 0.0)
    t = jnp.maximum(_dot(t, v2[...]) + b2[...], 0.0)
    t = _dot(t, v3[...]) + b3[...]
    out_ref[...] = _ln(t, lns[...], lnb[...]) + h_ref[...]


def _prep_body(h_ref, w_ref, b_ref, out_ref):
    out_ref[...] = _dot(h_ref[...], w_ref[0]) + b_ref[0]


def _full(shape):
    return pl.BlockSpec(shape, lambda *_: tuple(0 for _ in shape))


def _enc(x, p, rows, blk):
    (w1, b1), (w2, b2), (w3, b3) = p["layers"]
    in_dim = w1.shape[0]
    return pl.pallas_call(
        _enc_body,
        grid=(rows // blk,),
        in_specs=[
            pl.BlockSpec((blk, in_dim), lambda i: (i, 0)),
            _full(w1.shape), _full((1, LATENT)),
            _full(w2.shape), _full((1, LATENT)),
            _full(w3.shape), _full((1, LATENT)),
            _full((1, LATENT)), _full((1, LATENT)),
        ],
        out_specs=pl.BlockSpec((blk, LATENT), lambda i: (i, 0)),
        out_shape=jax.ShapeDtypeStruct((rows, LATENT), _f32),
    )(x, w1, b1.reshape(1, -1), w2, b2.reshape(1, -1), w3, b3.reshape(1, -1),
      p["ln_scale"].reshape(1, -1), p["ln_bias"].reshape(1, -1))


def _dec(h, p):
    (w1, b1), (w2, b2), (w3, b3) = p["layers"]
    return pl.pallas_call(
        _dec_body,
        grid=(N_NODES // R_N,),
        in_specs=[
            pl.BlockSpec((R_N, LATENT), lambda i: (i, 0)),
            _full(w1.shape), _full((1, LATENT)),
            _full(w2.shape), _full((1, LATENT)),
            _full(w3.shape), _full((1, OUT_DIM)),
        ],
        out_specs=pl.BlockSpec((R_N, OUT_DIM), lambda i: (i, 0)),
        out_shape=jax.ShapeDtypeStruct((N_NODES, OUT_DIM), _f32),
    )(h, w1, b1.reshape(1, -1), w2, b2.reshape(1, -1), w3, b3.reshape(1, -1))


def _prep(h, w_stack, b_stack):
    nb = N_NODES // R_N
    return pl.pallas_call(
        _prep_body,
        grid=(2, nb),
        in_specs=[
            pl.BlockSpec((R_N, LATENT), lambda i, j: (j, 0)),
            pl.BlockSpec((1, LATENT, LATENT), lambda i, j: (i, 0, 0)),
            pl.BlockSpec((1, 1, LATENT), lambda i, j: (i, 0, 0)),
        ],
        out_specs=pl.BlockSpec((R_N, LATENT), lambda i, j: (i * nb + j, 0)),
        out_shape=jax.ShapeDtypeStruct((2 * N_NODES, LATENT), _f32),
    )(h, w_stack, b_stack)


def _edge_mlp(e, g3, w1e, w2, b2, w3, b3, lns, lnb):
    return pl.pallas_call(
        _edge_body,
        grid=(N_EDGES // R_E,),
        in_specs=[
            pl.BlockSpec((R_E, LATENT), lambda i: (i, 0)),
            pl.BlockSpec((2, R_E, LATENT), lambda i: (0, i, 0)),
            _full((LATENT, LATENT)), _full((LATENT, LATENT)), _full((1, LATENT)),
            _full((LATENT, LATENT)), _full((1, LATENT)),
            _full((1, LATENT)), _full((1, LATENT)),
        ],
        out_specs=pl.BlockSpec((R_E, LATENT), lambda i: (i, 0)),
        out_shape=jax.ShapeDtypeStruct((N_EDGES, LATENT), _f32),
    )(e, g3, w1e, w2, b2.reshape(1, -1), w3, b3.reshape(1, -1),
      lns.reshape(1, -1), lnb.reshape(1, -1))


def _node_mlp(h, agg, v1h, v1a, b1, v2, b2, v3, b3, lns, lnb):
    return pl.pallas_call(
        _node_body,
        grid=(N_NODES // R_N,),
        in_specs=[
            pl.BlockSpec((R_N, LATENT), lambda i: (i, 0)),
            pl.BlockSpec((R_N, LATENT), lambda i: (i, 0)),
            _full((LATENT, LATENT)), _full((LATENT, LATENT)), _full((1, LATENT)),
            _full((LATENT, LATENT)), _full((1, LATENT)),
            _full((LATENT, LATENT)), _full((1, LATENT)),
            _full((1, LATENT)), _full((1, LATENT)),
        ],
        out_specs=pl.BlockSpec((R_N, LATENT), lambda i: (i, 0)),
        out_shape=jax.ShapeDtypeStruct((N_NODES, LATENT), _f32),
    )(h, agg, v1h, v1a, b1.reshape(1, -1), v2, b2.reshape(1, -1),
      v3, b3.reshape(1, -1), lns.reshape(1, -1), lnb.reshape(1, -1))


NB = 4


@functools.cache
def _get_gather_k():
    mesh = plsc.VectorSubcoreMesh(core_axis_name="c", subcore_axis_name="s",
                                  num_cores=NC, num_subcores=NS)
    scratch = ([pltpu.VMEM((CHUNK,), jnp.int32) for _ in range(NB)]
               + [pltpu.VMEM((CHUNK, LATENT), _f32) for _ in range(NB)]
               + [pltpu.SemaphoreType.DMA for _ in range(3 * NB)])

    @functools.partial(
        pl.kernel,
        mesh=mesh,
        out_type=jax.ShapeDtypeStruct((2 * N_EDGES, LATENT), _f32),
        scratch_types=scratch,
    )
    def _gather_k(table_hbm, idx_hbm, out_hbm, *bufs):
        idxb = bufs[:NB]
        rows = bufs[NB:2 * NB]
        si = bufs[2 * NB:3 * NB]
        sg = bufs[3 * NB:4 * NB]
        sw = bufs[4 * NB:5 * NB]
        wid = lax.axis_index("s") * NC + lax.axis_index("c")
        n_groups = (2 * N_EDGES) // CHUNK // NB
        base_n = n_groups // NW
        extra = n_groups - base_n * NW
        n_mine = base_n + jnp.where(wid < extra, 1, 0)

        def body(u, carry):
            g0 = (wid + NW * u) * NB
            di = [pltpu.async_copy(
                idx_hbm.at[pl.ds((g0 + b) * CHUNK, CHUNK)], idxb[b], si[b])
                for b in range(NB)]
            dg = []
            for b in range(NB):
                di[b].wait()
                dg.append(pltpu.async_copy(table_hbm.at[idxb[b]], rows[b], sg[b]))
            dw = []
            for b in range(NB):
                dg[b].wait()
                dw.append(pltpu.async_copy(
                    rows[b], out_hbm.at[pl.ds((g0 + b) * CHUNK, CHUNK)], sw[b]))
            for b in range(NB):
                dw[b].wait()
            return carry

        lax.fori_loop(0, n_mine, body, 0)

    return _gather_k


@functools.cache
def _get_scatter_k():
    mesh = plsc.VectorSubcoreMesh(core_axis_name="c", subcore_axis_name="s",
                                  num_cores=NC, num_subcores=NS)

    scratch = ([pltpu.VMEM((CHUNK,), jnp.int32) for _ in range(NB)]
               + [pltpu.VMEM((CHUNK, LATENT), _f32) for _ in range(NB)]
               + [pltpu.VMEM_SHARED((HALF + 8, LATENT), _f32)]
               + [pltpu.SemaphoreType.DMA for _ in range(3 * NB)])

    @functools.partial(
        pl.kernel,
        mesh=mesh,
        out_type=jax.ShapeDtypeStruct((NC, HALF, LATENT), _f32),
        scratch_types=scratch,
    )
    def _scatter_k(e_hbm, dst_hbm, z_hbm, out_hbm, *bufs):
        idxb = bufs[:NB]
        rows = bufs[NB:2 * NB]
        acc_sh = bufs[2 * NB]
        si = bufs[2 * NB + 1:2 * NB + 1 + NB]
        sr = bufs[2 * NB + 1 + NB:2 * NB + 1 + 2 * NB]
        sa = bufs[2 * NB + 1 + 2 * NB:2 * NB + 1 + 3 * NB]
        c = lax.axis_index("c")
        s = lax.axis_index("s")
        lo = c * HALF
        stripe = 312
        pltpu.sync_copy(z_hbm.at[pl.ds(s * stripe, stripe)],
                        acc_sh.at[pl.ds(s * stripe, stripe)])

        @pl.when(s == NS - 1)
        def _():
            pltpu.sync_copy(z_hbm.at[pl.ds(NS * stripe, 16)],
                            acc_sh.at[pl.ds(NS * stripe, 16)])

        plsc.subcore_barrier()

        n_groups = N_EDGES // CHUNK // NB
        base_n = n_groups // NS
        extra = n_groups - base_n * NS
        n_mine = base_n + jnp.where(s < extra, 1, 0)

        def body(u, carry):
            g0 = (s + NS * u) * NB
            di = [pltpu.async_copy(
                dst_hbm.at[pl.ds((g0 + b) * CHUNK, CHUNK)], idxb[b], si[b])
                for b in range(NB)]
            dr = [pltpu.async_copy(
                e_hbm.at[pl.ds((g0 + b) * CHUNK, CHUNK)], rows[b], sr[b])
                for b in range(NB)]
            da = []
            for b in range(NB):
                di[b].wait()
                for k in range(CHUNK // 16):
                    v = idxb[b][pl.ds(16 * k, 16)]
                    inb = (v >= lo) & (v < lo + HALF)
                    idxb[b][pl.ds(16 * k, 16)] = jnp.where(inb, v - lo, HALF)
                dr[b].wait()
                da.append(pltpu.async_copy(rows[b], acc_sh.at[idxb[b]], sa[b],
                                           add=True))
            for b in range(NB):
                da[b].wait()
            return carry

        lax.fori_loop(0, n_mine, body, 0)
        plsc.subcore_barrier()
        wtail = HALF - NS * stripe
        pltpu.sync_copy(acc_sh.at[pl.ds(s * stripe, stripe)],
                        out_hbm.at[c, pl.ds(s * stripe, stripe)])

        @pl.when(s == NS - 1)
        def _():
            pltpu.sync_copy(acc_sh.at[pl.ds(NS * stripe, wtail)],
                            out_hbm.at[c, pl.ds(NS * stripe, wtail)])

    return _scatter_k


def kernel(x, edge_index, edge_attr, params):
    src = edge_index[0]
    dst = edge_index[1]
    idx_all = jnp.concatenate([src, dst + N_NODES])
    zeros_nodes = jnp.zeros((N_NODES, LATENT), _f32)

    h = _enc(x, params["node_enc"], N_NODES, R_N)
    e = _enc(edge_attr, params["edge_enc"], N_EDGES, R_E)

    for layer in params["proc"]:
        ep = layer["edge"]
        (w1, b1), (w2, b2), (w3, b3) = ep["layers"]
        w1e = w1[:LATENT]
        w_stack = jnp.stack([w1[LATENT:2 * LATENT], w1[2 * LATENT:]])
        b_stack = jnp.stack([b1.reshape(1, -1), jnp.zeros((1, LATENT), _f32)])

        table = _prep(h, w_stack, b_stack)
        g3 = _get_gather_k()(table, idx_all).reshape(2, N_EDGES, LATENT)
        e = _edge_mlp(e, g3, w1e, w2, b2, w3, b3, ep["ln_scale"], ep["ln_bias"])

        agg = _get_scatter_k()(e, dst, zeros_nodes).reshape(N_NODES, LATENT)

        np_ = layer["node"]
        (v1, nb1), (v2, nb2), (v3, nb3) = np_["layers"]
        h = _node_mlp(h, agg, v1[:LATENT], v1[LATENT:], nb1, v2, nb2,
                      v3, nb3, np_["ln_scale"], np_["ln_bias"])

    return _dec(h, params["dec"])


# speedup vs baseline: 3.7160x; 1.0271x over previous
import functools

import jax
import jax.numpy as jnp
from jax import lax
from jax.experimental import pallas as pl
from jax.experimental.pallas import tpu as pltpu
from jax.experimental.pallas import tpu_sc as plsc

N_NODES = 10000
N_EDGES = 320000
LATENT = 128
OUT_DIM = 4

NC = 2
NS = 16
NW = NC * NS
CHUNK = 128
HALF = N_NODES // 2

R_E = 6400
R_N = 1000

_f32 = jnp.float32


def _ln(t, lns, lnb):
    mu = jnp.mean(t, axis=-1, keepdims=True)
    var = jnp.mean((t - mu) ** 2, axis=-1, keepdims=True)
    return (t - mu) * lax.rsqrt(var + 1e-5) * lns + lnb


def _dot(a, b):
    return jnp.dot(a, b, preferred_element_type=_f32)


def _enc_body(x_ref, w1, b1, w2, b2, w3, b3, lns, lnb, out_ref):
    t = jnp.maximum(_dot(x_ref[...], w1[...]) + b1[...], 0.0)
    t = jnp.maximum(_dot(t, w2[...]) + b2[...], 0.0)
    t = _dot(t, w3[...]) + b3[...]
    out_ref[...] = _ln(t, lns[...], lnb[...])


def _dec_body(x_ref, w1, b1, w2, b2, w3, b3, out_ref):
    t = jnp.maximum(_dot(x_ref[...], w1[...]) + b1[...], 0.0)
    t = jnp.maximum(_dot(t, w2[...]) + b2[...], 0.0)
    out_ref[...] = _dot(t, w3[...]) + b3[...]


def _edge_body(e_ref, g_ref, w1e, w2, b2, w3, b3, lns, lnb, out_ref):
    t = _dot(e_ref[...], w1e[...]) + g_ref[0] + g_ref[1]
    t = jnp.maximum(t, 0.0)
    t = jnp.maximum(_dot(t, w2[...]) + b2[...], 0.0)
    t = _dot(t, w3[...]) + b3[...]
    out_ref[...] = _ln(t, lns[...], lnb[...]) + e_ref[...]


def _node_body(h_ref, agg_ref, v1h, v1a, b1, v2, b2, v3, b3, lns, lnb, out_ref):
    t = _dot(h_ref[...], v1h[...]) + _dot(agg_ref[...], v1a[...]) + b1[...]
    t = jnp.maximum(t, 0.0)
    t = jnp.maximum(_dot(t, v2[...]) + b2[...], 0.0)
    t = _dot(t, v3[...]) + b3[...]
    out_ref[...] = _ln(t, lns[...], lnb[...]) + h_ref[...]


def _prep_body(h_ref, w_ref, b_ref, out_ref):
    out_ref[...] = _dot(h_ref[...], w_ref[0]) + b_ref[0]


def _full(shape):
    return pl.BlockSpec(shape, lambda *_: tuple(0 for _ in shape))


def _enc(x, p, rows, blk):
    (w1, b1), (w2, b2), (w3, b3) = p["layers"]
    in_dim = w1.shape[0]
    return pl.pallas_call(
        _enc_body,
        grid=(rows // blk,),
        in_specs=[
            pl.BlockSpec((blk, in_dim), lambda i: (i, 0)),
            _full(w1.shape), _full((1, LATENT)),
            _full(w2.shape), _full((1, LATENT)),
            _full(w3.shape), _full((1, LATENT)),
            _full((1, LATENT)), _full((1, LATENT)),
        ],
        out_specs=pl.BlockSpec((blk, LATENT), lambda i: (i, 0)),
        out_shape=jax.ShapeDtypeStruct((rows, LATENT), _f32),
    )(x, w1, b1.reshape(1, -1), w2, b2.reshape(1, -1), w3, b3.reshape(1, -1),
      p["ln_scale"].reshape(1, -1), p["ln_bias"].reshape(1, -1))


def _dec(h, p):
    (w1, b1), (w2, b2), (w3, b3) = p["layers"]
    return pl.pallas_call(
        _dec_body,
        grid=(N_NODES // R_N,),
        in_specs=[
            pl.BlockSpec((R_N, LATENT), lambda i: (i, 0)),
            _full(w1.shape), _full((1, LATENT)),
            _full(w2.shape), _full((1, LATENT)),
            _full(w3.shape), _full((1, OUT_DIM)),
        ],
        out_specs=pl.BlockSpec((R_N, OUT_DIM), lambda i: (i, 0)),
        out_shape=jax.ShapeDtypeStruct((N_NODES, OUT_DIM), _f32),
    )(h, w1, b1.reshape(1, -1), w2, b2.reshape(1, -1), w3, b3.reshape(1, -1))


def _prep(h, w_stack, b_stack):
    nb = N_NODES // R_N
    return pl.pallas_call(
        _prep_body,
        grid=(2, nb),
        in_specs=[
            pl.BlockSpec((R_N, LATENT), lambda i, j: (j, 0)),
            pl.BlockSpec((1, LATENT, LATENT), lambda i, j: (i, 0, 0)),
            pl.BlockSpec((1, 1, LATENT), lambda i, j: (i, 0, 0)),
        ],
        out_specs=pl.BlockSpec((R_N, LATENT), lambda i, j: (i * nb + j, 0)),
        out_shape=jax.ShapeDtypeStruct((2 * N_NODES, LATENT), _f32),
    )(h, w_stack, b_stack)


def _edge_mlp(e, g3, w1e, w2, b2, w3, b3, lns, lnb):
    return pl.pallas_call(
        _edge_body,
        grid=(N_EDGES // R_E,),
        in_specs=[
            pl.BlockSpec((R_E, LATENT), lambda i: (i, 0)),
            pl.BlockSpec((2, R_E, LATENT), lambda i: (0, i, 0)),
            _full((LATENT, LATENT)), _full((LATENT, LATENT)), _full((1, LATENT)),
            _full((LATENT, LATENT)), _full((1, LATENT)),
            _full((1, LATENT)), _full((1, LATENT)),
        ],
        out_specs=pl.BlockSpec((R_E, LATENT), lambda i: (i, 0)),
        out_shape=jax.ShapeDtypeStruct((N_EDGES, LATENT), _f32),
    )(e, g3, w1e, w2, b2.reshape(1, -1), w3, b3.reshape(1, -1),
      lns.reshape(1, -1), lnb.reshape(1, -1))


def _node_mlp(h, agg, v1h, v1a, b1, v2, b2, v3, b3, lns, lnb):
    return pl.pallas_call(
        _node_body,
        grid=(N_NODES // R_N,),
        in_specs=[
            pl.BlockSpec((R_N, LATENT), lambda i: (i, 0)),
            pl.BlockSpec((R_N, LATENT), lambda i: (i, 0)),
            _full((LATENT, LATENT)), _full((LATENT, LATENT)), _full((1, LATENT)),
            _full((LATENT, LATENT)), _full((1, LATENT)),
            _full((LATENT, LATENT)), _full((1, LATENT)),
            _full((1, LATENT)), _full((1, LATENT)),
        ],
        out_specs=pl.BlockSpec((R_N, LATENT), lambda i: (i, 0)),
        out_shape=jax.ShapeDtypeStruct((N_NODES, LATENT), _f32),
    )(h, agg, v1h, v1a, b1.reshape(1, -1), v2, b2.reshape(1, -1),
      v3, b3.reshape(1, -1), lns.reshape(1, -1), lnb.reshape(1, -1))


NB = 4


@functools.cache
def _get_gather_k():
    mesh = plsc.VectorSubcoreMesh(core_axis_name="c", subcore_axis_name="s",
                                  num_cores=NC, num_subcores=NS)
    scratch = ([pltpu.VMEM((CHUNK,), jnp.int32) for _ in range(NB)]
               + [pltpu.VMEM((CHUNK, LATENT), _f32) for _ in range(NB)]
               + [pltpu.SemaphoreType.DMA for _ in range(3 * NB)])

    @functools.partial(
        pl.kernel,
        mesh=mesh,
        out_type=jax.ShapeDtypeStruct((2 * N_EDGES, LATENT), _f32),
        scratch_types=scratch,
    )
    def _gather_k(table_hbm, idx_hbm, out_hbm, *bufs):
        idxb = bufs[:NB]
        rows = bufs[NB:2 * NB]
        si = bufs[2 * NB:3 * NB]
        sg = bufs[3 * NB:4 * NB]
        sw = bufs[4 * NB:5 * NB]
        wid = lax.axis_index("s") * NC + lax.axis_index("c")
        n_groups = (2 * N_EDGES) // CHUNK // NB
        base_n = n_groups // NW
        extra = n_groups - base_n * NW
        n_mine = base_n + jnp.where(wid < extra, 1, 0)

        def body(u, carry):
            g0 = (wid + NW * u) * NB
            di = [pltpu.async_copy(
                idx_hbm.at[pl.ds((g0 + b) * CHUNK, CHUNK)], idxb[b], si[b])
                for b in range(NB)]
            dg = []
            for b in range(NB):
                di[b].wait()
                dg.append(pltpu.async_copy(table_hbm.at[idxb[b]], rows[b], sg[b]))
            dw = []
            for b in range(NB):
                dg[b].wait()
                dw.append(pltpu.async_copy(
                    rows[b], out_hbm.at[pl.ds((g0 + b) * CHUNK, CHUNK)], sw[b]))
            for b in range(NB):
                dw[b].wait()
            return carry

        lax.fori_loop(0, n_mine, body, 0)

    return _gather_k


@functools.cache
def _get_scatter_k():
    mesh = plsc.VectorSubcoreMesh(core_axis_name="c", subcore_axis_name="s",
                                  num_cores=NC, num_subcores=NS)

    scratch = ([pltpu.VMEM((CHUNK,), jnp.int32) for _ in range(NB)]
               + [pltpu.VMEM((CHUNK, LATENT), _f32) for _ in range(NB)]
               + [pltpu.VMEM_SHARED((HALF + 8, LATENT), _f32)]
               + [pltpu.SemaphoreType.DMA for _ in range(3 * NB)])

    @functools.partial(
        pl.kernel,
        mesh=mesh,
        out_type=jax.ShapeDtypeStruct((NC, HALF, LATENT), _f32),
        scratch_types=scratch,
    )
    def _scatter_k(e_hbm, dst_hbm, z_hbm, out_hbm, *bufs):
        idxb = bufs[:NB]
        rows = bufs[NB:2 * NB]
        acc_sh = bufs[2 * NB]
        si = bufs[2 * NB + 1:2 * NB + 1 + NB]
        sr = bufs[2 * NB + 1 + NB:2 * NB + 1 + 2 * NB]
        sa = bufs[2 * NB + 1 + 2 * NB:2 * NB + 1 + 3 * NB]
        c = lax.axis_index("c")
        s = lax.axis_index("s")
        lo = c * HALF
        stripe = 312
        pltpu.sync_copy(z_hbm.at[pl.ds(s * stripe, stripe)],
                        acc_sh.at[pl.ds(s * stripe, stripe)])

        @pl.when(s == NS - 1)
        def _():
            pltpu.sync_copy(z_hbm.at[pl.ds(NS * stripe, 16)],
                            acc_sh.at[pl.ds(NS * stripe, 16)])

        plsc.subcore_barrier()

        n_groups = N_EDGES // CHUNK // NB
        base_n = n_groups // NS
        extra = n_groups - base_n * NS
        n_mine = base_n + jnp.where(s < extra, 1, 0)

        def body(u, carry):
            g0 = (s + NS * u) * NB
            di = [pltpu.async_copy(
                dst_hbm.at[pl.ds((g0 + b) * CHUNK, CHUNK)], idxb[b], si[b])
                for b in range(NB)]
            dr = [pltpu.async_copy(
                e_hbm.at[pl.ds((g0 + b) * CHUNK, CHUNK)], rows[b], sr[b])
                for b in range(NB)]
            da = []
            for b in range(NB):
                di[b].wait()
                for k in range(CHUNK // 16):
                    v = idxb[b][pl.ds(16 * k, 16)]
                    inb = (v >= lo) & (v < lo + HALF)
                    idxb[b][pl.ds(16 * k, 16)] = jnp.where(inb, v - lo, HALF)
                dr[b].wait()
                da.append(pltpu.async_copy(rows[b], acc_sh.at[idxb[b]], sa[b],
                                           add=True))
            for b in range(NB):
                da[b].wait()
            return carry

        lax.fori_loop(0, n_mine, body, 0)
        plsc.subcore_barrier()
        wtail = HALF - NS * stripe
        pltpu.sync_copy(acc_sh.at[pl.ds(s * stripe, stripe)],
                        out_hbm.at[c, pl.ds(s * stripe, stripe)])

        @pl.when(s == NS - 1)
        def _():
            pltpu.sync_copy(acc_sh.at[pl.ds(NS * stripe, wtail)],
                            out_hbm.at[c, pl.ds(NS * stripe, wtail)])

    return _scatter_k


def kernel(x, edge_index, edge_attr, params):
    src = edge_index[0]
    dst = edge_index[1]
    idx_all = jnp.concatenate([src, dst + N_NODES])
    zeros_nodes = jnp.zeros((N_NODES, LATENT), _f32)

    h = _enc(x, params["node_enc"], N_NODES, R_N)
    e = _enc(edge_attr, params["edge_enc"], N_EDGES, R_E)

    for layer in params["proc"]:
        ep = layer["edge"]
        (w1, b1), (w2, b2), (w3, b3) = ep["layers"]
        w1e = w1[:LATENT]
        w_stack = jnp.stack([w1[LATENT:2 * LATENT], w1[2 * LATENT:]])
        b_stack = jnp.stack([b1.reshape(1, -1), jnp.zeros((1, LATENT), _f32)])

        table = _prep(h, w_stack, b_stack)
        g3 = _get_gather_k()(table, idx_all).reshape(2, N_EDGES, LATENT)
        e = _edge_mlp(e, g3, w1e, w2, b2, w3, b3, ep["ln_scale"], ep["ln_bias"])

        agg = _get_scatter_k()(e, dst, zeros_nodes).reshape(N_NODES, LATENT)

        np_ = layer["node"]
        (v1, nb1), (v2, nb2), (v3, nb3) = np_["layers"]
        h = _node_mlp(h, agg, v1[:LATENT], v1[LATENT:], nb1, v2, nb2,
                      v3, nb3, np_["ln_scale"], np_["ln_bias"])

    return _dec(h, params["dec"])
